# baseline (speedup 1.0000x reference)
import functools

import numpy as np
import jax
import jax.numpy as jnp
from jax import lax
from jax.experimental import pallas as pl
from jax.experimental.pallas import tpu as pltpu
from jax.experimental.pallas import tpu_sc as plsc

_B, _L, _D = 16, 8192, 128
_ROWS = _B * _L
_NW = 32
_RPW = _ROWS // _NW
_CHUNK = 128
_NCH = _RPW // _CHUNK


def _flat_indices() -> np.ndarray:
    mseq = np.arange(_L)
    out = np.empty((_B, _L), dtype=np.int32)
    for i in range(_B):
        np.random.seed(i)
        out[i] = np.random.permutation(mseq) + i * _L
    return out.reshape(_NW * _NCH, _CHUNK)


_IDX = _flat_indices()

_mesh = plsc.VectorSubcoreMesh(core_axis_name="c", subcore_axis_name="s")


_UROWS = 128
_SPU = _UROWS // _CHUNK
_NSLOT = 2
_NU = _RPW // _UROWS
_NG = _NU // _NSLOT


@functools.partial(
    pl.kernel,
    out_type=jax.ShapeDtypeStruct((_ROWS, _D), jnp.float32),
    mesh=_mesh,
    scratch_types=[
        pltpu.VMEM((_NCH, _CHUNK), jnp.int32),
        pltpu.VMEM((_NSLOT, _UROWS, _D), jnp.float32),
        pltpu.VMEM_SHARED((16, _NSLOT, _UROWS, _D), jnp.float32),
        pltpu.SemaphoreType.DMA,
        pltpu.SemaphoreType.DMA,
        pltpu.SemaphoreType.DMA,
        pltpu.SemaphoreType.DMA,
        pltpu.SemaphoreType.DMA,
        pltpu.SemaphoreType.DMA,
    ],
)
def _interleave(x_hbm, idx_hbm, out_hbm, idx_v, buf, shr,
                g0, g1, c0, c1, w0, w1):
    sid = lax.axis_index("s")
    wid = sid * 2 + lax.axis_index("c")
    pltpu.sync_copy(idx_hbm.at[pl.ds(wid * _NCH, _NCH)], idx_v)
    base = wid * _RPW
    gsem = (g0, g1)
    csem = (c0, c1)
    wsem = (w0, w1)

    def start_gathers(u, r):
        for s in range(_SPU):
            pltpu.async_copy(
                x_hbm.at[idx_v.at[u * _SPU + s]],
                buf.at[r, pl.ds(s * _CHUNK, _CHUNK)],
                gsem[r],
            )

    def drain_gathers(r):
        pltpu.make_async_copy(x_hbm.at[pl.ds(0, _UROWS)], buf.at[r], gsem[r]).wait()

    def start_stage(r):
        pltpu.async_copy(buf.at[r], shr.at[sid, r], csem[r])

    def drain_stage(r):
        pltpu.make_async_copy(x_hbm.at[pl.ds(0, _UROWS)], shr.at[sid, r], csem[r]).wait()

    def start_write(u, r):
        pltpu.async_copy(shr.at[sid, r], out_hbm.at[pl.ds(base + u * _UROWS, _UROWS)], wsem[r])

    def drain_write(r):
        pltpu.make_async_copy(shr.at[sid, r], out_hbm.at[pl.ds(0, _UROWS)], wsem[r]).wait()

    @pl.loop(0, _NG)
    def _grp(g):
        for r in range(_NSLOT):
            @pl.when(g > 0)
            def _(r=r):
                drain_write(r)

            start_gathers(g * _NSLOT + r, r)
        for r in range(_NSLOT):
            drain_gathers(r)
            start_stage(r)
        for r in range(_NSLOT):
            drain_stage(r)
            start_write(g * _NSLOT + r, r)

    for r in range(_NSLOT):
        drain_write(r)


def kernel(x):
    b, L, d = x.shape
    flat = _interleave(x.reshape(_ROWS, _D), jnp.asarray(_IDX))
    return flat.reshape(b, L, d)


# speedup vs baseline: 1.0462x; 1.0462x over previous
import functools

import numpy as np
import jax
import jax.numpy as jnp
from jax import lax
from jax.experimental import pallas as pl
from jax.experimental.pallas import tpu as pltpu
from jax.experimental.pallas import tpu_sc as plsc

_B, _L, _D = 16, 8192, 128
_ROWS = _B * _L
_NW = 32
_RPW = _ROWS // _NW
_CHUNK = 128
_NCH = _RPW // _CHUNK


def _flat_indices() -> np.ndarray:
    mseq = np.arange(_L)
    out = np.empty((_B, _L), dtype=np.int32)
    for i in range(_B):
        np.random.seed(i)
        out[i] = np.random.permutation(mseq) + i * _L
    return out.reshape(_NW * _NCH, _CHUNK)


_IDX = _flat_indices()

_mesh = plsc.VectorSubcoreMesh(core_axis_name="c", subcore_axis_name="s")


_UROWS = 128
_SPU = _UROWS // _CHUNK
_NSLOT = 4
_NU = _RPW // _UROWS
_NG = _NU // _NSLOT


@functools.partial(
    pl.kernel,
    out_type=jax.ShapeDtypeStruct((_ROWS, _D), jnp.float32),
    mesh=_mesh,
    scratch_types=[
        pltpu.VMEM((_NCH, _CHUNK), jnp.int32),
        pltpu.VMEM((_NSLOT, _UROWS, _D), jnp.float32),
        pltpu.SemaphoreType.DMA,
        pltpu.SemaphoreType.DMA,
        pltpu.SemaphoreType.DMA,
        pltpu.SemaphoreType.DMA,
        pltpu.SemaphoreType.DMA,
        pltpu.SemaphoreType.DMA,
        pltpu.SemaphoreType.DMA,
        pltpu.SemaphoreType.DMA,
    ],
)
def _interleave(x_hbm, idx_hbm, out_hbm, idx_v, buf,
                g0, g1, g2, g3, w0, w1, w2, w3):
    wid = lax.axis_index("c") * 16 + lax.axis_index("s")
    pltpu.sync_copy(idx_hbm.at[pl.ds(wid * _NCH, _NCH)], idx_v)
    base = wid * _RPW
    gsem = (g0, g1, g2, g3)
    wsem = (w0, w1, w2, w3)

    def start_gathers(u, r):
        for s in range(_SPU):
            pltpu.async_copy(
                x_hbm.at[idx_v.at[u * _SPU + s]],
                buf.at[r, pl.ds(s * _CHUNK, _CHUNK)],
                gsem[r],
            )

    def drain_gathers(r):
        pltpu.make_async_copy(x_hbm.at[pl.ds(0, _UROWS)], buf.at[r], gsem[r]).wait()

    def start_write(u, r):
        pltpu.async_copy(buf.at[r], out_hbm.at[pl.ds(base + u * _UROWS, _UROWS)], wsem[r])

    def drain_write(r):
        pltpu.make_async_copy(buf.at[r], out_hbm.at[pl.ds(0, _UROWS)], wsem[r]).wait()

    @pl.loop(0, _NG)
    def _grp(g):
        for r in range(_NSLOT):
            @pl.when(g > 0)
            def _(r=r):
                drain_write(r)

            start_gathers(g * _NSLOT + r, r)
        for r in range(_NSLOT):
            drain_gathers(r)
            start_write(g * _NSLOT + r, r)

    for r in range(_NSLOT):
        drain_write(r)


def kernel(x):
    b, L, d = x.shape
    flat = _interleave(x.reshape(_ROWS, _D), jnp.asarray(_IDX))
    return flat.reshape(b, L, d)
